# baseline (speedup 1.0000x reference)
import functools

import jax
import jax.numpy as jnp
from jax import lax
from jax.experimental import pallas as pl
from jax.experimental.pallas import tpu as pltpu
from jax.experimental.pallas import tpu_sc as plsc

T = 1000
D = 128
B = 16384
NC = 2
NS = 16
NW = NC * NS
B_PER_W = B // NW
CH = 128
NCHUNK = B_PER_W // CH

_mesh = plsc.VectorSubcoreMesh(core_axis_name="c", subcore_axis_name="s")


@functools.partial(
    pl.kernel,
    mesh=_mesh,
    out_type=jax.ShapeDtypeStruct((B, D), jnp.float32),
    scratch_types=[
        pltpu.VMEM((NCHUNK, CH), jnp.int32),
        pltpu.VMEM((NCHUNK, CH, D), jnp.float32),
        [pltpu.SemaphoreType.DMA] * NCHUNK,
        pltpu.SemaphoreType.DMA,
    ],
)
def _gather_kernel(idx_hbm, table_hbm, out_hbm, idx_v, rows_v, gsems, psem):
    wid = lax.axis_index("s") * NC + lax.axis_index("c")
    base = wid * B_PER_W
    pltpu.sync_copy(idx_hbm.at[wid], idx_v)
    gathers = [
        pltpu.async_copy(table_hbm.at[idx_v.at[c]], rows_v.at[c], gsems[c])
        for c in range(NCHUNK)
    ]
    puts = []
    for c in range(NCHUNK):
        gathers[c].wait()
        puts.append(
            pltpu.async_copy(
                rows_v.at[c], out_hbm.at[pl.ds(base + c * CH, CH)], psem
            )
        )
    for p in puts:
        p.wait()


def kernel(t_idx, emb):
    idx = t_idx.astype(jnp.int32).reshape(NW, NCHUNK, CH)
    return _gather_kernel(idx, emb)


# speedup vs baseline: 1.1773x; 1.1773x over previous
import functools

import jax
import jax.numpy as jnp
from jax import lax
from jax.experimental import pallas as pl
from jax.experimental.pallas import tpu as pltpu
from jax.experimental.pallas import tpu_sc as plsc

T = 1000
D = 128
B = 16384
NC = 2
NS = 16
NW = NC * NS
B_PER_W = B // NW
STAGE_TILES = 8
STAGE_ROWS = 128

_mesh = plsc.VectorSubcoreMesh(core_axis_name="c", subcore_axis_name="s")


@functools.partial(
    pl.kernel,
    mesh=_mesh,
    out_type=jax.ShapeDtypeStruct((B, D), jnp.float32),
    scratch_types=[
        pltpu.VMEM((B_PER_W,), jnp.int32),
        pltpu.VMEM((B_PER_W, D), jnp.float32),
        pltpu.VMEM_SHARED((T, D), jnp.float32),
        pltpu.SemaphoreType.DMA,
    ],
)
def _gather_kernel(idx_hbm, table_hbm, out_hbm, idx_v, rows_v, table_sh, sem):
    sid = lax.axis_index("s")
    wid = sid * NC + lax.axis_index("c")
    base = wid * B_PER_W
    idx_cp = pltpu.async_copy(idx_hbm.at[pl.ds(base, B_PER_W)], idx_v, sem)

    for k in range(STAGE_TILES):
        nrows = min(STAGE_ROWS, T - k * STAGE_ROWS)

        @pl.when(sid == k)
        def _stage(k=k, nrows=nrows):
            pltpu.sync_copy(
                table_hbm.at[pl.ds(k * STAGE_ROWS, nrows)],
                table_sh.at[pl.ds(k * STAGE_ROWS, nrows)],
            )

    idx_cp.wait()
    plsc.subcore_barrier()
    pltpu.async_copy(table_sh.at[idx_v], rows_v, sem).wait()
    pltpu.sync_copy(rows_v, out_hbm.at[pl.ds(base, B_PER_W)])


def kernel(t_idx, emb):
    return _gather_kernel(t_idx.astype(jnp.int32), emb)


# speedup vs baseline: 1.2101x; 1.0279x over previous
import functools

import jax
import jax.numpy as jnp
from jax import lax
from jax.experimental import pallas as pl
from jax.experimental.pallas import tpu as pltpu
from jax.experimental.pallas import tpu_sc as plsc

T = 1000
D = 128
B = 16384
NC = 2
NS = 16
NW = NC * NS
B_PER_W = B // NW
CH = 128
NCHUNK = B_PER_W // CH
STAGE_TILES = 8
STAGE_ROWS = 128

_mesh = plsc.VectorSubcoreMesh(core_axis_name="c", subcore_axis_name="s")


@functools.partial(
    pl.kernel,
    mesh=_mesh,
    out_type=jax.ShapeDtypeStruct((B, D), jnp.float32),
    scratch_types=[
        pltpu.VMEM((NCHUNK, CH), jnp.int32),
        pltpu.VMEM((NCHUNK, CH, D), jnp.float32),
        pltpu.VMEM_SHARED((T, D), jnp.float32),
        [pltpu.SemaphoreType.DMA] * NCHUNK,
        pltpu.SemaphoreType.DMA,
    ],
)
def _gather_kernel(idx_hbm, table_hbm, out_hbm, idx_v, rows_v, table_sh, gsems, psem):
    sid = lax.axis_index("s")
    wid = sid * NC + lax.axis_index("c")
    base = wid * B_PER_W
    idx_cp = pltpu.async_copy(idx_hbm.at[wid], idx_v, psem)

    for k in range(STAGE_TILES):
        nrows = min(STAGE_ROWS, T - k * STAGE_ROWS)

        @pl.when(sid == k)
        def _stage(k=k, nrows=nrows):
            pltpu.sync_copy(
                table_hbm.at[pl.ds(k * STAGE_ROWS, nrows)],
                table_sh.at[pl.ds(k * STAGE_ROWS, nrows)],
            )

    idx_cp.wait()
    plsc.subcore_barrier()
    gathers = [
        pltpu.async_copy(table_sh.at[idx_v.at[c]], rows_v.at[c], gsems[c])
        for c in range(NCHUNK)
    ]
    puts = []
    for c in range(NCHUNK):
        gathers[c].wait()
        puts.append(
            pltpu.async_copy(
                rows_v.at[c], out_hbm.at[pl.ds(base + c * CH, CH)], psem
            )
        )
    for p in puts:
        p.wait()


def kernel(t_idx, emb):
    idx = t_idx.astype(jnp.int32).reshape(NW, NCHUNK, CH)
    return _gather_kernel(idx, emb)


# speedup vs baseline: 1.2334x; 1.0192x over previous
import functools

import jax
import jax.numpy as jnp
from jax import lax
from jax.experimental import pallas as pl
from jax.experimental.pallas import tpu as pltpu
from jax.experimental.pallas import tpu_sc as plsc

T = 1000
D = 128
B = 16384
NC = 2
NS = 16
NW = NC * NS
B_PER_W = B // NW
CH = 64
NCHUNK = B_PER_W // CH
STAGE_TILES = 8
STAGE_ROWS = 128

_mesh = plsc.VectorSubcoreMesh(core_axis_name="c", subcore_axis_name="s")


@functools.partial(
    pl.kernel,
    mesh=_mesh,
    out_type=jax.ShapeDtypeStruct((B, D), jnp.float32),
    scratch_types=[
        pltpu.VMEM((NCHUNK, CH), jnp.int32),
        pltpu.VMEM((NCHUNK, CH, D), jnp.float32),
        pltpu.VMEM_SHARED((T, D), jnp.float32),
        [pltpu.SemaphoreType.DMA] * NCHUNK,
        pltpu.SemaphoreType.DMA,
    ],
)
def _gather_kernel(idx_hbm, table_hbm, out_hbm, idx_v, rows_v, table_sh, gsems, psem):
    sid = lax.axis_index("s")
    wid = sid * NC + lax.axis_index("c")
    base = wid * B_PER_W
    idx_cp = pltpu.async_copy(idx_hbm.at[wid], idx_v, psem)

    for k in range(STAGE_TILES):
        nrows = min(STAGE_ROWS, T - k * STAGE_ROWS)

        @pl.when(sid == k)
        def _stage(k=k, nrows=nrows):
            pltpu.sync_copy(
                table_hbm.at[pl.ds(k * STAGE_ROWS, nrows)],
                table_sh.at[pl.ds(k * STAGE_ROWS, nrows)],
            )

    plsc.subcore_barrier()
    idx_cp.wait()
    gathers = [
        pltpu.async_copy(table_sh.at[idx_v.at[c]], rows_v.at[c], gsems[c])
        for c in range(NCHUNK)
    ]
    puts = []
    for c in range(NCHUNK):
        gathers[c].wait()
        puts.append(
            pltpu.async_copy(
                rows_v.at[c], out_hbm.at[pl.ds(base + c * CH, CH)], psem
            )
        )
    for p in puts:
        p.wait()


def kernel(t_idx, emb):
    idx = t_idx.astype(jnp.int32).reshape(NW, NCHUNK, CH)
    return _gather_kernel(idx, emb)
